# Initial kernel scaffold; baseline (speedup 1.0000x reference)
import jax
import jax.numpy as jnp
from jax.experimental import pallas as pl


def kernel(coords, obses, actions, W_coord, W_field, W_action):
    raise NotImplementedError("write your pallas kernel here")


# speedup vs baseline: 3.6882x; 3.6882x over previous
import functools

import jax
import jax.numpy as jnp
from jax import lax
from jax.experimental import pallas as pl
from jax.experimental.pallas import tpu as pltpu
from jax.experimental.pallas import tpu_sc as plsc

B = 16384
RF2 = 121
CD = 64
FD = RF2 * 16
AD = 2
OUT = CD + FD + AD
NC, NS = 2, 16
NW = NC * NS
R = B // NW
C = 16
NCHUNK = R // C

_mesh = plsc.VectorSubcoreMesh(core_axis_name="c", subcore_axis_name="s")


@functools.partial(
    pl.kernel,
    out_type=jax.ShapeDtypeStruct((B, OUT), jnp.float32),
    mesh=_mesh,
    compiler_params=pltpu.CompilerParams(use_tc_tiling_on_sc=False,
                                         needs_layout_passes=False),
    scratch_types=[
        pltpu.VMEM((C * RF2,), jnp.int32),
        pltpu.VMEM((C * 4,), jnp.int32),
        pltpu.VMEM((C,), jnp.int32),
        pltpu.VMEM((C * RF2, 16), jnp.float32),
        pltpu.VMEM((C * 4, 16), jnp.float32),
        pltpu.VMEM((C, OUT), jnp.float32),
        pltpu.VMEM((4, AD), jnp.float32),
        pltpu.SemaphoreType.DMA,
        pltpu.SemaphoreType.DMA,
    ],
)
def _hint_kernel(w16, wf, wa, cidx_hbm, fidx_hbm, act_hbm, out,
                 fidx_v, cidx_v, act_v, fbuf, cbuf, obuf, wa_v, sem1, sem2):
    wid = lax.axis_index("s") * NC + lax.axis_index("c")
    pltpu.sync_copy(wa, wa_v)

    @pl.loop(0, NCHUNK)
    def _chunk(g):
        base = wid * R + g * C
        pltpu.sync_copy(fidx_hbm.at[pl.ds(base * RF2, C * RF2)], fidx_v)
        pltpu.sync_copy(cidx_hbm.at[pl.ds(base * 4, C * 4)], cidx_v)
        pltpu.sync_copy(act_hbm.at[pl.ds(base, C)], act_v)
        df = pltpu.async_copy(wf.at[fidx_v], fbuf, sem1)
        dc = pltpu.async_copy(w16.at[cidx_v], cbuf, sem2)

        av = act_v[...]
        zeros = jnp.zeros((16,), jnp.int32)
        ones = jnp.ones((16,), jnp.int32)
        w0 = plsc.load_gather(wa_v, [av, zeros])
        w1 = plsc.load_gather(wa_v, [av, ones])
        rows = lax.iota(jnp.int32, 16)
        plsc.store_scatter(obuf, [rows, zeros + (CD + FD)], w0)
        plsc.store_scatter(obuf, [rows, zeros + (CD + FD + 1)], w1)

        dc.wait()
        df.wait()

        @pl.loop(0, C)
        def _row(r):
            for j in range(4):
                obuf[r, pl.ds(16 * j, 16)] = cbuf[r * 4 + j, :]
            for j in range(RF2):
                obuf[r, pl.ds(CD + 16 * j, 16)] = fbuf[r * RF2 + j, :]

        pltpu.sync_copy(obuf, out.at[pl.ds(base, C), :])


def kernel(coords, obses, actions, W_coord, W_field, W_action):
    c2 = coords.astype(jnp.int32) * 2
    cidx = jnp.stack([c2[:, 0], c2[:, 0] + 1, c2[:, 1], c2[:, 1] + 1],
                     axis=1).reshape(-1)
    fidx = obses.astype(jnp.int32).reshape(-1)
    act = actions.astype(jnp.int32).reshape(-1)
    w16 = W_coord.reshape(2000, 16)
    return _hint_kernel(w16, W_field, W_action, cidx, fidx, act)


# speedup vs baseline: 4.1503x; 1.1253x over previous
import functools

import jax
import jax.numpy as jnp
from jax import lax
from jax.experimental import pallas as pl
from jax.experimental.pallas import tpu as pltpu
from jax.experimental.pallas import tpu_sc as plsc

B = 16384
RF2 = 121
CD = 64
FD = RF2 * 16
AD = 2
OUT = CD + FD + AD
NC, NS = 2, 16
NW = NC * NS
R = B // NW
C = 8
NCHUNK = R // C

_mesh = plsc.VectorSubcoreMesh(core_axis_name="c", subcore_axis_name="s")


@functools.partial(
    pl.kernel,
    out_type=jax.ShapeDtypeStruct((B, OUT), jnp.float32),
    mesh=_mesh,
    compiler_params=pltpu.CompilerParams(use_tc_tiling_on_sc=False,
                                         needs_layout_passes=False),
    scratch_types=[
        pltpu.VMEM((2, C * RF2), jnp.int32),
        pltpu.VMEM((R * 4,), jnp.int32),
        pltpu.VMEM((R,), jnp.int32),
        pltpu.VMEM((2, C * RF2, 16), jnp.float32),
        pltpu.VMEM((2, C * 4, 16), jnp.float32),
        pltpu.VMEM((2, C, OUT), jnp.float32),
        pltpu.VMEM((8,), jnp.float32),
        pltpu.SemaphoreType.DMA,
        pltpu.SemaphoreType.DMA,
        pltpu.SemaphoreType.DMA,
        pltpu.SemaphoreType.DMA,
        pltpu.SemaphoreType.DMA,
        pltpu.SemaphoreType.DMA,
        pltpu.SemaphoreType.DMA,
    ],
)
def _hint_kernel(w16, wf, wa, cidx_hbm, fidx_hbm, act_hbm, out,
                 fidx_v, cidx_v, act_v, fbuf, cbuf, obuf, wa_v,
                 semf0, semf1, semc0, semc1, semw0, semw1, sems):
    wid = lax.axis_index("s") * NC + lax.axis_index("c")
    rbase = wid * R
    pltpu.sync_copy(wa, wa_v)
    pltpu.sync_copy(cidx_hbm.at[pl.ds(rbase * 4, R * 4)], cidx_v)
    pltpu.sync_copy(act_hbm.at[pl.ds(rbase, R)], act_v)

    semf = (semf0, semf1)
    semc = (semc0, semc1)
    semw = (semw0, semw1)

    def load_and_fire(g, s):
        base = rbase + g * C
        pltpu.async_copy(fidx_hbm.at[pl.ds(base * RF2, C * RF2)],
                         fidx_v.at[s], sems).wait()
        fd = pltpu.make_async_copy(wf.at[fidx_v.at[s]], fbuf.at[s], semf[s])
        fd.start()
        cd = pltpu.make_async_copy(w16.at[cidx_v.at[pl.ds(g * C * 4, C * 4)]],
                                   cbuf.at[s], semc[s])
        cd.start()

    def process(g, t, s):
        base = rbase + g * C
        pltpu.make_async_copy(wf.at[fidx_v.at[s]], fbuf.at[s], semf[s]).wait()
        pltpu.make_async_copy(w16.at[cidx_v.at[pl.ds(g * C * 4, C * 4)]],
                              cbuf.at[s], semc[s]).wait()
        @pl.when(t >= 1)
        def _():
            pltpu.make_async_copy(obuf.at[s], out.at[pl.ds(base, C), :],
                                  semw[s]).wait()

        @pl.loop(0, C)
        def _row(r):
            for j in range(4):
                obuf[s, r, pl.ds(16 * j, 16)] = cbuf[s, r * 4 + j, :]
            for j in range(RF2):
                obuf[s, r, pl.ds(CD + 16 * j, 16)] = fbuf[s, r * RF2 + j, :]

        lanes = lax.iota(jnp.int32, 16)
        rows = lanes // 2
        cols = lanes % 2
        a = plsc.load_gather(act_v, [g * C + rows])
        w = plsc.load_gather(wa_v, [a * 2 + cols])
        plsc.store_scatter(obuf.at[s], [rows, cols + (CD + FD)], w)

        wd = pltpu.make_async_copy(obuf.at[s], out.at[pl.ds(base, C), :], semw[s])
        wd.start()

    load_and_fire(0, 0)

    @pl.loop(0, NCHUNK // 2)
    def _pair(t):
        g0 = 2 * t
        load_and_fire(g0 + 1, 1)
        process(g0, t, 0)

        @pl.when(g0 + 2 < NCHUNK)
        def _():
            load_and_fire(g0 + 2, 0)

        process(g0 + 1, t, 1)

    pltpu.make_async_copy(obuf.at[0], out.at[pl.ds(rbase, C), :], semw0).wait()
    pltpu.make_async_copy(obuf.at[1], out.at[pl.ds(rbase, C), :], semw1).wait()


def kernel(coords, obses, actions, W_coord, W_field, W_action):
    c2 = coords.astype(jnp.int32) * 2
    cidx = jnp.stack([c2[:, 0], c2[:, 0] + 1, c2[:, 1], c2[:, 1] + 1],
                     axis=1).reshape(-1)
    fidx = obses.astype(jnp.int32).reshape(-1)
    act = actions.astype(jnp.int32).reshape(-1)
    w16 = W_coord.reshape(2000, 16)
    wa = W_action.reshape(-1)
    return _hint_kernel(w16, W_field, wa, cidx, fidx, act)
